# Initial kernel scaffold; baseline (speedup 1.0000x reference)
import jax
import jax.numpy as jnp
from jax.experimental import pallas as pl


def kernel(token_ids, bigram_emb_weight, gate):
    raise NotImplementedError("write your pallas kernel here")


# speedup vs baseline: 1.3557x; 1.3557x over previous
import functools

import jax
import jax.numpy as jnp
from jax import lax
from jax.experimental import pallas as pl
from jax.experimental.pallas import tpu as pltpu
from jax.experimental.pallas import tpu_sc as plsc

_B, _T, _D = 4, 4096, 1024
_HASH = 32768
_MULT_MOD = 1000003 % _HASH
_N = _B * _T
_L = 16

_NW = 32
_ROWS_PER_W = _N // _NW
_CHUNK = 32
_NCHUNK = _ROWS_PER_W // _CHUNK
_NBUF = 3


def _tec_body(tok_hbm, gate_hbm, table_hbm, out_hbm,
              tok_v, key_v, g_v, rows_v,
              gsem0, gsem1, gsem2, osem0, osem1, osem2):
    wid = lax.axis_index("s") * 2 + lax.axis_index("c")
    base = wid * _ROWS_PER_W

    pltpu.sync_copy(tok_hbm.at[pl.ds(base, _ROWS_PER_W)],
                    tok_v.at[pl.ds(_L, _ROWS_PER_W)])

    @pl.when(wid > 0)
    def _():
        pltpu.sync_copy(tok_hbm.at[pl.ds(base - _L, _L)],
                        tok_v.at[pl.ds(0, _L)])

    pltpu.sync_copy(gate_hbm, g_v)
    for d in range(_D // _L):
        x = g_v[pl.ds(d * _L, _L)]
        g_v[pl.ds(d * _L, _L)] = 1.0 / (1.0 + jnp.exp(-x))

    lanes = lax.iota(jnp.int32, _L)
    lane0 = 1 - jnp.minimum(lanes, 1)
    is_start = jnp.int32(1) - jnp.minimum(jnp.int32(base % _T), 1)
    keep0 = 1 - lane0 * is_start
    for j in range(_ROWS_PER_W // _L):
        tok = tok_v[pl.ds(_L + j * _L, _L)]
        prev = tok_v[pl.ds(_L - 1 + j * _L, _L)]
        if j == 0:
            prev = prev * keep0
        key = ((prev & (_HASH - 1)) * _MULT_MOD + tok) & (_HASH - 1)
        key_v[pl.ds(j * _L, _L)] = key

    gsems = [gsem0, gsem1, gsem2]
    osems = [osem0, osem1, osem2]
    gh, oh = {}, {}

    def start_gather(c):
        idx = key_v.at[pl.ds(c * _CHUNK, _CHUNK)]
        gh[c] = pltpu.async_copy(table_hbm.at[idx], rows_v.at[c % _NBUF],
                                 gsems[c % _NBUF])

    def scale_chunk(buf):
        for grp in range(_D // (16 * _L)):
            gs = tuple(g_v[pl.ds(grp * 16 * _L + t * _L, _L)]
                       for t in range(16))

            def scale_rows(r, gs):
                for t in range(16):
                    off = grp * 16 * _L + t * _L
                    buf[r, pl.ds(off, _L)] = buf[r, pl.ds(off, _L)] * gs[t]
                return gs

            lax.fori_loop(0, _CHUNK, scale_rows, gs)

    start_gather(0)
    start_gather(1)
    for c in range(_NCHUNK):
        gh[c].wait()
        buf = rows_v.at[c % _NBUF]
        scale_chunk(buf)
        oh[c] = pltpu.async_copy(
            buf, out_hbm.at[pl.ds(base + c * _CHUNK, _CHUNK)],
            osems[c % _NBUF])
        n = c + 2
        if n < _NCHUNK:
            if n - _NBUF >= 0:
                oh[n - _NBUF].wait()
            start_gather(n)
    for c in range(_NCHUNK - _NBUF, _NCHUNK):
        oh[c].wait()


@functools.partial(jax.jit, static_argnames=())
def _run(tok_flat, gate, table):
    mesh = plsc.VectorSubcoreMesh(core_axis_name="c", subcore_axis_name="s")
    k = pl.kernel(
        _tec_body,
        mesh=mesh,
        out_type=jax.ShapeDtypeStruct((_N, _D), jnp.float32),
        scratch_types=[
            pltpu.VMEM((_ROWS_PER_W + _L,), jnp.int32),
            pltpu.VMEM((_ROWS_PER_W,), jnp.int32),
            pltpu.VMEM((_D,), jnp.float32),
            pltpu.VMEM((_NBUF, _CHUNK, _D), jnp.float32),
            pltpu.SemaphoreType.DMA,
            pltpu.SemaphoreType.DMA,
            pltpu.SemaphoreType.DMA,
            pltpu.SemaphoreType.DMA,
            pltpu.SemaphoreType.DMA,
            pltpu.SemaphoreType.DMA,
        ],
    )
    return k(tok_flat, gate, table)


def kernel(token_ids, bigram_emb_weight, gate):
    out = _run(token_ids.reshape(-1), gate, bigram_emb_weight)
    return out.reshape(_B, _T, _D)


# speedup vs baseline: 1.5044x; 1.1097x over previous
import functools

import jax
import jax.numpy as jnp
from jax import lax
from jax.experimental import pallas as pl
from jax.experimental.pallas import tpu as pltpu
from jax.experimental.pallas import tpu_sc as plsc

_B, _T, _D = 4, 4096, 1024
_HASH = 32768
_MULT_MOD = 1000003 % _HASH
_N = _B * _T
_L = 16

_NW = 32
_ROWS_PER_W = _N // _NW
_CHUNK = 32
_NCHUNK = _ROWS_PER_W // _CHUNK
_NBUF = 3


def _tec_body(tok_hbm, gate_hbm, table_hbm, out_hbm,
              tok_v, g_v, rows_v, gsem, osem):
    wid = lax.axis_index("s") * 2 + lax.axis_index("c")
    base = wid * _ROWS_PER_W

    pltpu.sync_copy(tok_hbm.at[pl.ds(base, _ROWS_PER_W)],
                    tok_v.at[pl.ds(_L, _ROWS_PER_W)])

    @pl.when(wid > 0)
    def _():
        pltpu.sync_copy(tok_hbm.at[pl.ds(base - _L, _L)],
                        tok_v.at[pl.ds(0, _L)])

    pltpu.sync_copy(gate_hbm, g_v)
    for d in range(_D // _L):
        x = g_v[pl.ds(d * _L, _L)]
        g_v[pl.ds(d * _L, _L)] = 1.0 / (1.0 + jnp.exp(-x))

    lanes = lax.iota(jnp.int32, _L)
    lane0 = 1 - jnp.minimum(lanes, 1)
    is_start = jnp.int32(1) - jnp.minimum(jnp.int32(base % _T), 1)
    keep0 = 1 - lane0 * is_start
    for j in reversed(range(_ROWS_PER_W // _L)):
        tok = tok_v[pl.ds(_L + j * _L, _L)]
        prev = tok_v[pl.ds(_L - 1 + j * _L, _L)]
        if j == 0:
            prev = prev * keep0
        key = ((prev & (_HASH - 1)) * _MULT_MOD + tok) & (_HASH - 1)
        tok_v[pl.ds(_L + j * _L, _L)] = key

    def key_slice(c):
        return tok_v.at[pl.ds(_L + c * _CHUNK, _CHUNK)]

    def start_gather(c, b):
        pltpu.async_copy(table_hbm.at[key_slice(c)], rows_v.at[b],
                         gsem.at[b])

    def scale_chunk(buf):
        for grp in range(_D // (16 * _L)):
            gs = tuple(g_v[pl.ds(grp * 16 * _L + t * _L, _L)]
                       for t in range(16))

            def scale_rows(r, gs):
                for t in range(16):
                    off = grp * 16 * _L + t * _L
                    buf[r, pl.ds(off, _L)] = buf[r, pl.ds(off, _L)] * gs[t]
                return gs

            lax.fori_loop(0, _CHUNK, scale_rows, gs)

    start_gather(0, 0)
    start_gather(1, 1)

    def chunk_body(c, _):
        b = lax.rem(c, _NBUF)
        pltpu.make_async_copy(table_hbm.at[pl.ds(0, _CHUNK)], rows_v.at[b],
                              gsem.at[b]).wait()
        scale_chunk(rows_v.at[b])
        pltpu.async_copy(rows_v.at[b],
                         out_hbm.at[pl.ds(base + c * _CHUNK, _CHUNK)],
                         osem.at[b])
        n = c + 2
        bn = lax.rem(n, _NBUF)

        @pl.when(n < _NCHUNK)
        def _():
            @pl.when(n >= _NBUF)
            def _():
                pltpu.make_async_copy(rows_v.at[bn],
                                      out_hbm.at[pl.ds(0, _CHUNK)],
                                      osem.at[bn]).wait()
            start_gather(n, bn)
        return 0

    lax.fori_loop(0, _NCHUNK, chunk_body, 0)
    for c in range(_NCHUNK - _NBUF, _NCHUNK):
        pltpu.make_async_copy(rows_v.at[c % _NBUF],
                              out_hbm.at[pl.ds(0, _CHUNK)],
                              osem.at[c % _NBUF]).wait()


@functools.partial(jax.jit, static_argnames=())
def _run(tok_flat, gate, table):
    mesh = plsc.VectorSubcoreMesh(core_axis_name="c", subcore_axis_name="s")
    k = pl.kernel(
        _tec_body,
        mesh=mesh,
        out_type=jax.ShapeDtypeStruct((_N, _D), jnp.float32),
        scratch_types=[
            pltpu.VMEM((_ROWS_PER_W + _L,), jnp.int32),
            pltpu.VMEM((_D,), jnp.float32),
            pltpu.VMEM((_NBUF, _CHUNK, _D), jnp.float32),
            pltpu.SemaphoreType.DMA((_NBUF,)),
            pltpu.SemaphoreType.DMA((_NBUF,)),
        ],
    )
    return k(tok_flat, gate, table)


def kernel(token_ids, bigram_emb_weight, gate):
    out = _run(token_ids.reshape(-1), gate, bigram_emb_weight)
    return out.reshape(_B, _T, _D)


# speedup vs baseline: 1.5452x; 1.0271x over previous
import functools

import jax
import jax.numpy as jnp
from jax import lax
from jax.experimental import pallas as pl
from jax.experimental.pallas import tpu as pltpu
from jax.experimental.pallas import tpu_sc as plsc

_B, _T, _D = 4, 4096, 1024
_HASH = 32768
_MULT_MOD = 1000003 % _HASH
_N = _B * _T
_L = 16

_NW = 32
_ROWS_PER_W = _N // _NW
_CHUNK = 16
_NCHUNK = _ROWS_PER_W // _CHUNK
_NBUF = 6
_GDEPTH = 3


def _tec_body(tok_hbm, gate_hbm, table_hbm, out_hbm,
              tok_v, g_v, rows_v, gsem, osem):
    wid = lax.axis_index("s") * 2 + lax.axis_index("c")
    base = wid * _ROWS_PER_W

    pltpu.sync_copy(tok_hbm.at[pl.ds(base, _ROWS_PER_W)],
                    tok_v.at[pl.ds(_L, _ROWS_PER_W)])

    @pl.when(wid > 0)
    def _():
        pltpu.sync_copy(tok_hbm.at[pl.ds(base - _L, _L)],
                        tok_v.at[pl.ds(0, _L)])

    pltpu.sync_copy(gate_hbm, g_v)
    for d in range(_D // _L):
        x = g_v[pl.ds(d * _L, _L)]
        g_v[pl.ds(d * _L, _L)] = 1.0 / (1.0 + jnp.exp(-x))

    lanes = lax.iota(jnp.int32, _L)
    lane0 = 1 - jnp.minimum(lanes, 1)
    is_start = jnp.int32(1) - jnp.minimum(jnp.int32(base % _T), 1)
    keep0 = 1 - lane0 * is_start
    for j in reversed(range(_ROWS_PER_W // _L)):
        tok = tok_v[pl.ds(_L + j * _L, _L)]
        prev = tok_v[pl.ds(_L - 1 + j * _L, _L)]
        if j == 0:
            prev = prev * keep0
        key = ((prev & (_HASH - 1)) * _MULT_MOD + tok) & (_HASH - 1)
        tok_v[pl.ds(_L + j * _L, _L)] = key

    def key_slice(c):
        return tok_v.at[pl.ds(_L + c * _CHUNK, _CHUNK)]

    def start_gather(c, b):
        pltpu.async_copy(table_hbm.at[key_slice(c)], rows_v.at[b],
                         gsem.at[b])

    def scale_chunk(buf):
        for grp in range(_D // (16 * _L)):
            gs = tuple(g_v[pl.ds(grp * 16 * _L + t * _L, _L)]
                       for t in range(16))

            def scale_rows(r, gs):
                for t in range(16):
                    off = grp * 16 * _L + t * _L
                    buf[r, pl.ds(off, _L)] = buf[r, pl.ds(off, _L)] * gs[t]
                return gs

            lax.fori_loop(0, _CHUNK, scale_rows, gs)

    for c0 in range(_GDEPTH):
        start_gather(c0, c0)

    def chunk_body(c, _):
        b = lax.rem(c, _NBUF)
        pltpu.make_async_copy(table_hbm.at[pl.ds(0, _CHUNK)], rows_v.at[b],
                              gsem.at[b]).wait()
        scale_chunk(rows_v.at[b])
        pltpu.async_copy(rows_v.at[b],
                         out_hbm.at[pl.ds(base + c * _CHUNK, _CHUNK)],
                         osem.at[b])
        n = c + _GDEPTH
        bn = lax.rem(n, _NBUF)

        @pl.when(n < _NCHUNK)
        def _():
            @pl.when(n >= _NBUF)
            def _():
                pltpu.make_async_copy(rows_v.at[bn],
                                      out_hbm.at[pl.ds(0, _CHUNK)],
                                      osem.at[bn]).wait()
            start_gather(n, bn)
        return 0

    lax.fori_loop(0, _NCHUNK, chunk_body, 0)
    for c in range(_NCHUNK - _NBUF, _NCHUNK):
        pltpu.make_async_copy(rows_v.at[c % _NBUF],
                              out_hbm.at[pl.ds(0, _CHUNK)],
                              osem.at[c % _NBUF]).wait()


@functools.partial(jax.jit, static_argnames=())
def _run(tok_flat, gate, table):
    mesh = plsc.VectorSubcoreMesh(core_axis_name="c", subcore_axis_name="s")
    k = pl.kernel(
        _tec_body,
        mesh=mesh,
        out_type=jax.ShapeDtypeStruct((_N, _D), jnp.float32),
        scratch_types=[
            pltpu.VMEM((_ROWS_PER_W + _L,), jnp.int32),
            pltpu.VMEM((_D,), jnp.float32),
            pltpu.VMEM((_NBUF, _CHUNK, _D), jnp.float32),
            pltpu.SemaphoreType.DMA((_NBUF,)),
            pltpu.SemaphoreType.DMA((_NBUF,)),
        ],
    )
    return k(tok_flat, gate, table)


def kernel(token_ids, bigram_emb_weight, gate):
    out = _run(token_ids.reshape(-1), gate, bigram_emb_weight)
    return out.reshape(_B, _T, _D)


# speedup vs baseline: 1.5457x; 1.0003x over previous
import functools

import jax
import jax.numpy as jnp
from jax import lax
from jax.experimental import pallas as pl
from jax.experimental.pallas import tpu as pltpu
from jax.experimental.pallas import tpu_sc as plsc

_B, _T, _D = 4, 4096, 1024
_HASH = 32768
_MULT_MOD = 1000003 % _HASH
_N = _B * _T
_L = 16

_NW = 32
_ROWS_PER_W = _N // _NW
_CHUNK = 16
_NCHUNK = _ROWS_PER_W // _CHUNK
_NBUF = 6
_GDEPTH = 3


def _tec_body(tok_hbm, gate_hbm, table_hbm, out_hbm,
              tok_v, g_v, rows_v, gsem, osem):
    wid = lax.axis_index("s") * 2 + lax.axis_index("c")
    base = wid * _ROWS_PER_W

    pltpu.sync_copy(tok_hbm.at[pl.ds(base, _ROWS_PER_W)],
                    tok_v.at[pl.ds(_L, _ROWS_PER_W)])

    @pl.when(wid > 0)
    def _():
        pltpu.sync_copy(tok_hbm.at[pl.ds(base - _L, _L)],
                        tok_v.at[pl.ds(0, _L)])

    pltpu.sync_copy(gate_hbm, g_v)
    for d in range(_D // _L):
        x = g_v[pl.ds(d * _L, _L)]
        g_v[pl.ds(d * _L, _L)] = 1.0 / (1.0 + jnp.exp(-x))

    lanes = lax.iota(jnp.int32, _L)
    lane0 = 1 - jnp.minimum(lanes, 1)
    is_start = jnp.int32(1) - jnp.minimum(jnp.int32(base % _T), 1)
    keep0 = 1 - lane0 * is_start
    for j in reversed(range(_ROWS_PER_W // _L)):
        tok = tok_v[pl.ds(_L + j * _L, _L)]
        prev = tok_v[pl.ds(_L - 1 + j * _L, _L)]
        if j == 0:
            prev = prev * keep0
        key = ((prev & (_HASH - 1)) * _MULT_MOD + tok) & (_HASH - 1)
        tok_v[pl.ds(_L + j * _L, _L)] = key

    def key_slice(c):
        return tok_v.at[pl.ds(_L + c * _CHUNK, _CHUNK)]

    def start_gather(c, b):
        pltpu.async_copy(table_hbm.at[key_slice(c)], rows_v.at[b],
                         gsem.at[b])

    _G = 8

    def scale_chunk(buf):
        for grp in range(_D // (_G * _L)):
            gs = tuple(g_v[pl.ds(grp * _G * _L + t * _L, _L)]
                       for t in range(_G))

            def scale_rows(r, _):
                for t in range(_G):
                    off = grp * _G * _L + t * _L
                    buf[r, pl.ds(off, _L)] = buf[r, pl.ds(off, _L)] * gs[t]
                return 0

            lax.fori_loop(0, _CHUNK, scale_rows, 0)

    for c0 in range(_GDEPTH):
        start_gather(c0, c0)

    def chunk_body(c, _):
        b = lax.rem(c, _NBUF)
        pltpu.make_async_copy(table_hbm.at[pl.ds(0, _CHUNK)], rows_v.at[b],
                              gsem.at[b]).wait()
        scale_chunk(rows_v.at[b])
        pltpu.async_copy(rows_v.at[b],
                         out_hbm.at[pl.ds(base + c * _CHUNK, _CHUNK)],
                         osem.at[b])
        n = c + _GDEPTH
        bn = lax.rem(n, _NBUF)

        @pl.when(n < _NCHUNK)
        def _():
            @pl.when(n >= _NBUF)
            def _():
                pltpu.make_async_copy(rows_v.at[bn],
                                      out_hbm.at[pl.ds(0, _CHUNK)],
                                      osem.at[bn]).wait()
            start_gather(n, bn)
        return 0

    lax.fori_loop(0, _NCHUNK, chunk_body, 0)
    for c in range(_NCHUNK - _NBUF, _NCHUNK):
        pltpu.make_async_copy(rows_v.at[c % _NBUF],
                              out_hbm.at[pl.ds(0, _CHUNK)],
                              osem.at[c % _NBUF]).wait()


@functools.partial(jax.jit, static_argnames=())
def _run(tok_flat, gate, table):
    mesh = plsc.VectorSubcoreMesh(core_axis_name="c", subcore_axis_name="s")
    k = pl.kernel(
        _tec_body,
        mesh=mesh,
        out_type=jax.ShapeDtypeStruct((_N, _D), jnp.float32),
        scratch_types=[
            pltpu.VMEM((_ROWS_PER_W + _L,), jnp.int32),
            pltpu.VMEM((_D,), jnp.float32),
            pltpu.VMEM((_NBUF, _CHUNK, _D), jnp.float32),
            pltpu.SemaphoreType.DMA((_NBUF,)),
            pltpu.SemaphoreType.DMA((_NBUF,)),
        ],
    )
    return k(tok_flat, gate, table)


def kernel(token_ids, bigram_emb_weight, gate):
    out = _run(token_ids.reshape(-1), gate, bigram_emb_weight)
    return out.reshape(_B, _T, _D)
